# Initial kernel scaffold; baseline (speedup 1.0000x reference)
import jax
import jax.numpy as jnp
from jax.experimental import pallas as pl


def kernel(frame, time_stamp, frame_n, W_f, W_i, time_buf, poses_buf, fmap_buf, imap_buf, patches_buf, patch_state, source_frame, i_buf, j_buf, w_buf, v_buf):
    raise NotImplementedError("write your pallas kernel here")


# speedup vs baseline: 1.0154x; 1.0154x over previous
import jax
import jax.numpy as jnp
from jax.experimental import pallas as pl

R_MIN = 0.5
R_MAX = 30.0
FLS_H = 512
FLS_W = 512
FOV_H = 130.0
BUFF = 16
P = 256
PS = 8
T = 8
C = 64
DOWN = 4
FH = FLS_H // DOWN
FW = FLS_W // DOWN
SLOT = 2 * P * T
MAX_EDGES = BUFF * SLOT


def _hamilton(q1, q2):
    x1, y1, z1, w1 = q1[0], q1[1], q1[2], q1[3]
    x2, y2, z2, w2 = q2[0], q2[1], q2[2], q2[3]
    w = w1 * w2 - x1 * x2 - y1 * y2 - z1 * z2
    x = w1 * x2 + x1 * w2 + y1 * z2 - z1 * y2
    y = w1 * y2 - x1 * z2 + y1 * w2 + z1 * x2
    z = w1 * z2 + x1 * y2 - y1 * x2 + z1 * w2
    return jnp.stack([x, y, z, w])


def _encoder_body(xbT_ref, wfT_ref, wiT_ref, fmap_ref, imap_ref):
    xbT = xbT_ref[...]
    fmap_ref[...] = jnp.dot(wfT_ref[...], xbT, preferred_element_type=jnp.float32)
    imap_ref[...] = jnp.tanh(
        jnp.dot(wiT_ref[...], xbT, preferred_element_type=jnp.float32))


def kernel(frame, time_stamp, frame_n, W_f, W_i, time_buf, poses_buf,
           fmap_buf, imap_buf, patches_buf, patch_state, source_frame,
           i_buf, j_buf, w_buf, v_buf):
    frame_n = jnp.asarray(frame_n, dtype=jnp.int32)
    local = frame_n % BUFF

    x = frame[0, 0]
    xbT = x.reshape(FH, DOWN, FW, DOWN).transpose(1, 3, 0, 2).reshape(
        DOWN * DOWN, FH * FW)

    fmap_flat, imap_flat = pl.pallas_call(
        _encoder_body,
        out_shape=(
            jax.ShapeDtypeStruct((C, FH * FW), jnp.float32),
            jax.ShapeDtypeStruct((C, FH * FW), jnp.float32),
        ),
    )(xbT, W_f.T, W_i.T)
    fmap = fmap_flat.reshape(C, FH, FW)
    imap = imap_flat.reshape(C, FH, FW)

    score = (x * x).reshape(-1)
    _, idx = jax.lax.top_k(score, P)
    ys = idx // FLS_W
    xs = idx % FLS_W
    coords = jnp.stack([xs.astype(jnp.float32), ys.astype(jnp.float32)], axis=1)
    cy = jnp.clip(ys // DOWN - PS // 2, 0, FH - PS)
    cx = jnp.clip(xs // DOWN - PS // 2, 0, FW - PS)
    rows = cy[:, None] + jnp.arange(PS)
    cols = cx[:, None] + jnp.arange(PS)
    new_patches = fmap[:, rows[:, :, None], cols[:, None, :]]
    new_patches = new_patches.transpose(1, 0, 2, 3)

    fmap_buf = fmap_buf.at[local].set(fmap)
    imap_buf = imap_buf.at[local].set(imap)
    time_buf = time_buf.at[local].set(time_stamp[0])

    r_norm = coords[:, 1] / FLS_H
    r = r_norm * (R_MAX - R_MIN) + R_MIN
    theta_norm = coords[:, 0] / FLS_W - 0.5
    theta = theta_norm * FOV_H * jnp.pi / 180.0
    phi = jnp.zeros((P,), dtype=jnp.float32)
    patches_buf = patches_buf.at[local].set(new_patches)
    patch_state = patch_state.at[local].set(jnp.stack([r, theta, phi], axis=1))
    source_frame = source_frame.at[local].set(
        jnp.full((P,), frame_n, dtype=jnp.int32))

    k1 = (local - 1) % BUFF
    k2 = (local - 2) % BUFF
    t0 = time_buf[local]
    t1 = time_buf[k1]
    t2 = time_buf[k2]
    x1 = poses_buf[k1]
    x2 = poses_buf[k2]
    new_pose = x1[0:3] + (x1[0:3] - x2[0:3]) / (t1 - t2) * (t0 - t1)
    q1 = x1[3:]
    q2 = x2[3:]
    dot = (q1 * q2).sum()
    q1 = jnp.where(dot < 0, -q1, q1)
    diff = _hamilton(q1, jnp.concatenate([-q2[:3], q2[3:]]))
    s = jnp.sqrt(jnp.clip(1.0 - diff[3] * diff[3], 0.0))
    rot_axis = jnp.where(s < 1e-3,
                         jnp.array([1.0, 0.0, 0.0], dtype=jnp.float32),
                         diff[:3] / jnp.maximum(s, 1e-12))
    rot_angle = 2.0 * jnp.arccos(jnp.clip(diff[3], -1.0, 1.0))
    rot_a = rot_angle / (t1 - t2) * (t0 - t1)
    q_step = jnp.concatenate(
        [rot_axis * jnp.sin(rot_a / 2.0), jnp.cos(rot_a / 2.0)[None]])
    q0 = _hamilton(q_step, q1)
    q0 = q0 / jnp.linalg.norm(q0)
    x0 = jnp.concatenate([new_pose, q0])
    poses_buf = poses_buf.at[local].set(x0)

    new_patch_ids = frame_n * P + jnp.arange(P, dtype=jnp.int32)
    past_frames = frame_n - 1 - jnp.arange(T, dtype=jnp.int32)
    i_new = jnp.tile(new_patch_ids, T)
    j_past = jnp.repeat(past_frames, P)
    i_past = (frame_n - T) * P + jnp.arange(T * P, dtype=jnp.int32)
    j_cur = jnp.full((T * P,), frame_n, dtype=jnp.int32)
    new_i = jnp.concatenate([i_new, i_past])
    new_j = jnp.concatenate([j_past, j_cur])
    lo = local * SLOT
    i_buf = jax.lax.dynamic_update_slice(i_buf, new_i, (lo,))
    j_buf = jax.lax.dynamic_update_slice(j_buf, new_j, (lo,))
    w_buf = jax.lax.dynamic_update_slice(
        w_buf, jnp.zeros((SLOT,), dtype=jnp.float32), (lo,))
    v_buf = jax.lax.dynamic_update_slice(
        v_buf, jnp.ones((SLOT,), dtype=bool), (lo,))

    return (fmap_buf, imap_buf, patches_buf, patch_state, poses_buf, time_buf,
            source_frame, i_buf, j_buf, w_buf, v_buf)
